# baseline (speedup 1.0000x reference)
import functools

import jax
import jax.numpy as jnp
from jax import lax
from jax.experimental import pallas as pl
from jax.experimental.pallas import tpu as pltpu
from jax.experimental.pallas import tpu_sc as plsc

NUM_NOTES = 91
EMB = 8
OUT_D = 10
LANES = 16
NW = 32
NBUF = 3


@functools.lru_cache(maxsize=None)
def _build(N):
    n_per_w = N // NW
    C = 3200
    n_chunks = n_per_w // C

    mesh = plsc.VectorSubcoreMesh(core_axis_name="c", subcore_axis_name="s")

    @functools.partial(
        pl.kernel,
        mesh=mesh,
        out_type=jax.ShapeDtypeStruct((OUT_D, N), jnp.float32),
        scratch_types=[
            pltpu.VMEM((NUM_NOTES * EMB,), jnp.float32),
            [pltpu.VMEM((C,), jnp.int32) for _ in range(NBUF)],
            [pltpu.VMEM((EMB, C), jnp.float32) for _ in range(NBUF)],
            [pltpu.SemaphoreType.DMA for _ in range(NBUF)],
            [pltpu.SemaphoreType.DMA for _ in range(NBUF)],
            pltpu.SemaphoreType.DMA,
        ],
        compiler_params=pltpu.CompilerParams(
            needs_layout_passes=False, use_tc_tiling_on_sc=False
        ),
    )
    def k(tab_hbm, notes_hbm, on_hbm, dur_hbm, out_hbm,
          tab_v, notes_v, p_v, in_sem, out_sem, od_sem):
        wid = lax.axis_index("s") * 2 + lax.axis_index("c")
        wbase = pl.multiple_of(wid * n_per_w, C)
        od_h = [
            pltpu.async_copy(on_hbm.at[pl.ds(wbase, n_per_w)],
                             out_hbm.at[EMB, pl.ds(wbase, n_per_w)], od_sem),
            pltpu.async_copy(dur_hbm.at[pl.ds(wbase, n_per_w)],
                             out_hbm.at[EMB + 1, pl.ds(wbase, n_per_w)], od_sem),
        ]
        pltpu.sync_copy(tab_hbm, tab_v)
        in_h = [None] * NBUF
        out_h = [None] * NBUF

        def fire_in(g):
            b = g % NBUF
            base = pl.multiple_of(wid * n_per_w + g * C, C)
            in_h[b] = [
                pltpu.async_copy(notes_hbm.at[pl.ds(base, C)], notes_v[b], in_sem[b]),
            ]

        fire_in(0)
        fire_in(1)
        for g in range(n_chunks):
            b = g % NBUF
            for h in in_h[b]:
                h.wait()

            nv = notes_v[b]
            pv = p_v[b]

            @plsc.parallel_loop(0, C, step=LANES, unroll=4)
            def gat_body(i):
                off = pl.multiple_of(i, LANES)
                nt = nv[pl.ds(off, LANES)]
                for d in range(EMB):
                    e = plsc.load_gather(tab_v, [nt + (NUM_NOTES * d)])
                    pv[d, pl.ds(off, LANES)] = e

            base = pl.multiple_of(wid * n_per_w + g * C, C)
            out_h[b] = pltpu.async_copy(
                pv, out_hbm.at[pl.ds(0, EMB), pl.ds(base, C)], out_sem[b]
            )
            if g + 2 < n_chunks:
                if g >= 1:
                    out_h[(g + 2) % NBUF].wait()
                fire_in(g + 2)
        for g in (n_chunks - 3, n_chunks - 2, n_chunks - 1):
            out_h[g % NBUF].wait()
        for h in od_h:
            h.wait()

    return k


def _tile_order(x, L, B):
    return x.T.reshape(L // 8, 8, B // 128, 128).transpose(0, 2, 1, 3).reshape(L * B)


@jax.jit
def kernel(notes, onsets, durations, note_embedding_weight):
    B, L = notes.shape
    N = B * L
    tab_cm = note_embedding_weight.T.reshape(NUM_NOTES * EMB)
    notes_p = _tile_order(notes, L, B)
    on_p = _tile_order(onsets[:, :, 0], L, B)
    dur_p = _tile_order(durations[:, :, 0], L, B)
    out = _build(N)(tab_cm, notes_p, on_p, dur_p)
    out5 = out.reshape(OUT_D, L // 8, B // 128, 8, 128)
    return out5.transpose(2, 4, 1, 3, 0).reshape(B, L, OUT_D)


# speedup vs baseline: 4.6718x; 4.6718x over previous
import functools

import jax
import jax.numpy as jnp
from jax import lax
from jax.experimental import pallas as pl
from jax.experimental.pallas import tpu as pltpu
from jax.experimental.pallas import tpu_sc as plsc

NUM_NOTES = 91
EMB = 8
OUT_D = 10
LANES = 16
NW = 32
NBUF = 3


@functools.lru_cache(maxsize=None)
def _build(N):
    n_per_w = N // NW
    C = 3200
    n_chunks = n_per_w // C

    mesh = plsc.VectorSubcoreMesh(core_axis_name="c", subcore_axis_name="s")

    @functools.partial(
        pl.kernel,
        mesh=mesh,
        out_type=jax.ShapeDtypeStruct((OUT_D, N), jnp.float32),
        scratch_types=[
            pltpu.VMEM((NUM_NOTES * EMB,), jnp.float32),
            [pltpu.VMEM((C,), jnp.int32) for _ in range(NBUF)],
            [pltpu.VMEM((OUT_D, C), jnp.float32) for _ in range(NBUF)],
            [pltpu.SemaphoreType.DMA for _ in range(NBUF)],
            [pltpu.SemaphoreType.DMA for _ in range(NBUF)],
        ],
        compiler_params=pltpu.CompilerParams(
            needs_layout_passes=False, use_tc_tiling_on_sc=False
        ),
    )
    def k(tab_hbm, notes_hbm, on_hbm, dur_hbm, out_hbm,
          tab_v, notes_v, p_v, in_sem, out_sem):
        wid = lax.axis_index("s") * 2 + lax.axis_index("c")
        pltpu.sync_copy(tab_hbm, tab_v)
        in_h = [None] * NBUF
        out_h = [None] * NBUF

        def fire_in(g):
            b = g % NBUF
            base = pl.multiple_of(wid * n_per_w + g * C, C)
            in_h[b] = [
                pltpu.async_copy(notes_hbm.at[pl.ds(base, C)], notes_v[b], in_sem[b]),
                pltpu.async_copy(on_hbm.at[pl.ds(base, C)], p_v[b].at[EMB], in_sem[b]),
                pltpu.async_copy(dur_hbm.at[pl.ds(base, C)], p_v[b].at[EMB + 1], in_sem[b]),
            ]

        fire_in(0)
        fire_in(1)
        for g in range(n_chunks):
            b = g % NBUF
            for h in in_h[b]:
                h.wait()

            nv = notes_v[b]
            pv = p_v[b]

            @plsc.parallel_loop(0, C, step=LANES, unroll=4)
            def gat_body(i):
                off = pl.multiple_of(i, LANES)
                nt = nv[pl.ds(off, LANES)]
                for d in range(EMB):
                    e = plsc.load_gather(tab_v, [nt + (NUM_NOTES * d)])
                    pv[d, pl.ds(off, LANES)] = e

            base = pl.multiple_of(wid * n_per_w + g * C, C)
            out_h[b] = pltpu.async_copy(pv, out_hbm.at[:, pl.ds(base, C)], out_sem[b])
            if g + 2 < n_chunks:
                if g >= 1:
                    out_h[(g + 2) % NBUF].wait()
                fire_in(g + 2)
        for g in (n_chunks - 3, n_chunks - 2, n_chunks - 1):
            out_h[g % NBUF].wait()

    return k


def _tile_order(x, L, B):
    return x.T.reshape(L // 8, 8, B // 128, 128).transpose(0, 2, 1, 3).reshape(L * B)


@jax.jit
def kernel(notes, onsets, durations, note_embedding_weight):
    B, L = notes.shape
    N = B * L
    tab_cm = note_embedding_weight.T.reshape(NUM_NOTES * EMB)
    notes_p = _tile_order(notes, L, B)
    on_p = _tile_order(onsets[:, :, 0], L, B)
    dur_p = _tile_order(durations[:, :, 0], L, B)
    out = _build(N)(tab_cm, notes_p, on_p, dur_p)
    out5 = out.reshape(OUT_D, L // 8, B // 128, 8, 128)
    return out5.transpose(2, 4, 1, 3, 0).reshape(B, L, OUT_D)
